# baseline (speedup 1.0000x reference)
import functools

import jax
import jax.numpy as jnp
from jax import lax
from jax.experimental import pallas as pl
from jax.experimental.pallas import tpu as pltpu
from jax.experimental.pallas import tpu_sc as plsc

_L = 16
_INT_MIN = -2147483648
_MASKVAL = -32000.0


def _scores_body(q_ref, k_ref, o_ref):
    o_ref[0] = lax.dot_general(
        q_ref[0], k_ref[0], (((1,), (1,)), ((), ())),
        precision=lax.Precision.HIGHEST, preferred_element_type=jnp.float32)


def _scores(q, k):
    n, t, h = q.shape
    bm = 256
    return pl.pallas_call(
        _scores_body,
        grid=(n, t // bm),
        in_specs=[pl.BlockSpec((1, bm, h), lambda i, j: (i, j, 0)),
                  pl.BlockSpec((1, t, h), lambda i, j: (i, 0, 0))],
        out_specs=pl.BlockSpec((1, bm, t), lambda i, j: (i, j, 0)),
        out_shape=jax.ShapeDtypeStruct((n, t, t), jnp.float32),
    )(q, k)


def _iota():
    return lax.iota(jnp.int32, _L)


def _splat_i(x):
    return jnp.full((_L,), x, dtype=jnp.int32)


def _splat_f(x):
    return jnp.full((_L,), x, dtype=jnp.float32)


def _rhe(x):
    t = x.astype(jnp.int32)
    r = x - t.astype(jnp.float32)
    odd = (t & 1) == 1
    up = (r > 0.5) | ((r == 0.5) & odd)
    return t + up.astype(jnp.int32)


def _sortable(s):
    b = lax.bitcast_convert_type(s, jnp.int32)
    return jnp.where(b < 0, b ^ jnp.int32(0x7FFFFFFF), b)


def _sort16(v):
    b = jnp.int32(_INT_MIN)
    return lax.sort(v ^ b, dimension=0) ^ b


def _bmerge(vs):
    if len(vs) == 1:
        return [_sort16(vs[0])]
    half = len(vs) // 2
    lo = [jnp.minimum(vs[i], vs[i + half]) for i in range(half)]
    hi = [jnp.maximum(vs[i], vs[i + half]) for i in range(half)]
    return _bmerge(lo) + _bmerge(hi)


def _merge(a, b):
    rb = [lax.rev(x, (0,)) for x in reversed(b)]
    return _bmerge(a + rb)


def _sort128(vs):
    runs = [[_sort16(v)] for v in vs]
    while len(runs) > 1:
        runs = [_merge(runs[i], runs[i + 1]) for i in range(0, len(runs), 2)]
    return runs[0]


def _make_sc_attend(total_rows, t_src, h):
    info = plsc.get_sparse_core_info()
    nw = info.num_cores * info.num_subcores
    rows_per = total_rows // nw
    nch = 8
    ech = 17
    mesh = plsc.VectorSubcoreMesh(core_axis_name="c", subcore_axis_name="s")

    @functools.partial(
        pl.kernel, mesh=mesh,
        compiler_params=pltpu.CompilerParams(
            needs_layout_passes=False, use_tc_tiling_on_sc=False),
        out_type=jax.ShapeDtypeStruct((total_rows, h), jnp.float32),
        scratch_types=[
            pltpu.VMEM((t_src,), jnp.float32),
            pltpu.VMEM((128,), jnp.int32),
            pltpu.VMEM((144,), jnp.int32),
            pltpu.VMEM((ech * _L,), jnp.int32),
            pltpu.VMEM((ech * _L,), jnp.int32),
            pltpu.VMEM((16 + ech * _L,), jnp.int32),
            pltpu.VMEM((64,), jnp.int32),
            pltpu.VMEM((64, h), jnp.float32),
            pltpu.VMEM((h,), jnp.float32),
            pltpu.SemaphoreType.DMA,
        ],
    )
    def sc_attend(s_hbm, v_hbm, out_hbm, srow, pixbuf, key0buf,
                  embuf, incbuf, rmbuf, idxbuf, vrows, ctxbuf, sem):
        wid = lax.axis_index("c") * info.num_subcores + lax.axis_index("s")

        def row_body(r, _):
            row = r * nw + wid
            a = lax.rem(row, t_src)
            tsrc = a + 1
            tsrcf = _splat_f(tsrc.astype(jnp.float32))
            pltpu.sync_copy(s_hbm.at[row], srow)

            def level(it, carry):
                ps = list(carry[:nch])
                msplat = carry[nch]
                itv = _splat_i(it)
                is0 = itv == 0
                is5 = itv == 5
                wsbase = _splat_i(64 << it).astype(jnp.float32)
                ws = jnp.where(is0, _splat_f(64.0),
                               jnp.minimum(tsrcf, wsbase))
                ratio = tsrcf / ws
                factor = jnp.where(is5, _splat_f(1.0), _splat_f(1.5))
                zm1 = jnp.where(is0, _splat_i(95), _splat_i(127))
                cap = jnp.where(is5, _splat_i(64), _splat_i(128))
                th1_idx = jnp.where(is0, _splat_i(65),
                                    jnp.where(is5, _splat_i(64),
                                              _splat_i(32)))
                ws_new = jnp.minimum(tsrcf, ws * 2.0)
                scale = ws_new / ws

                xt = ws / tsrcf * 64.0 * factor
                bound = jnp.minimum((ws - 1.0).astype(jnp.int32), zm1)
                tks = jnp.minimum(jnp.maximum(_rhe(xt), 1), bound)

                io = _iota()
                key0s = []
                for c in range(nch):
                    zv = io + (c * _L)
                    pf = ps[c].astype(jnp.float32)
                    txs = jnp.minimum(_rhe(pf * ratio), t_src - 1)
                    sc = plsc.load_gather(srow, [txs])
                    sc = jnp.where(zv < msplat, sc, _splat_f(_MASKVAL))
                    k0 = _sortable(sc)
                    key0buf[pl.ds(_L + c * _L, _L)] = k0
                    key0s.append(k0)
                keys = []
                carry_rf = _splat_i(0)
                for c in range(nch):
                    zv = io + (c * _L)
                    prev = key0buf[pl.ds(_L - 1 + c * _L, _L)]
                    chg = (zv == 0) | (key0s[c] != prev)
                    tt = jnp.where(chg, zv, _splat_i(0))
                    rf = jnp.maximum(plsc.cummax(tt), carry_rf)
                    carry_rf = _splat_i(jnp.max(rf))
                    kk = key0s[c] - (zv - rf)
                    ph = is0 & (zv >= 96)
                    keys.append(jnp.where(ph, _splat_i(_INT_MIN), kk))

                skeys = _sort128(keys)
                th2_idx = jnp.minimum(_splat_i(128) - tks, 127)
                th1 = _splat_i(0)
                th2 = _splat_i(0)
                for c in range(nch):
                    zv = io + (c * _L)
                    th1 = th1 + jnp.where(zv == th1_idx, skeys[c],
                                          _splat_i(0))
                    th2 = th2 + jnp.where(zv == th2_idx, skeys[c],
                                          _splat_i(0))
                th1 = _splat_i(jnp.sum(th1))
                th2 = _splat_i(jnp.sum(th2))
                tks_pos = tks > 0

                cnt0 = _rhe(scale)
                inc_head = ((tks > msplat) & (cnt0 > 1)).astype(jnp.int32)
                io = _iota()
                embuf[pl.ds(0, _L)] = jnp.where(io == 1, 1, 0)
                head_inc = jnp.where(
                    io == 0, _splat_i(1),
                    jnp.where(io == 1, inc_head, _splat_i(0)))
                incbuf[pl.ds(0, _L)] = head_inc
                incbuf[pl.ds(256, _L)] = _splat_i(0)
                for c in range(nch):
                    zv = _iota() + (c * _L)
                    validc = zv < msplat
                    sel1 = (keys[c] >= th1) & validc
                    sel2 = (keys[c] >= th2) & tks_pos & validc
                    pf = ps[c].astype(jnp.float32)
                    c0 = _rhe(pf * scale)
                    pe = _rhe((pf + 1.0) * scale)
                    c1 = c0 + ((pe - c0 > 1) & sel2).astype(jnp.int32)
                    slot0 = 2 + 2 * zv
                    plsc.store_scatter(embuf, [slot0], c0)
                    plsc.store_scatter(embuf, [slot0 + 1], c1)
                    plsc.store_scatter(incbuf, [slot0],
                                       sel1.astype(jnp.int32))
                    plsc.store_scatter(incbuf, [slot0 + 1],
                                       sel2.astype(jnp.int32))

                for c in range(nch):
                    pixbuf[pl.ds(c * _L, _L)] = _splat_i(0)
                carry_rm = _splat_i(-1)
                carry_cs = _splat_i(0)
                for j in range(ech):
                    vals = embuf[pl.ds(j * _L, _L)]
                    incs = incbuf[pl.ds(j * _L, _L)]
                    mv = jnp.where(incs > 0, vals, _splat_i(-1))
                    rm = jnp.maximum(plsc.cummax(mv), carry_rm)
                    rmbuf[pl.ds(_L + j * _L, _L)] = rm
                    excl = rmbuf[pl.ds(_L - 1 + j * _L, _L)]
                    if j == 0:
                        excl = jnp.where(io == 0, _splat_i(-1), excl)
                    carry_rm = _splat_i(jnp.max(rm))
                    keep = (incs > 0) & (vals > excl)
                    ki = keep.astype(jnp.int32)
                    kc = plsc.cumsum(ki)
                    dest = carry_cs + kc - ki
                    plsc.store_scatter(pixbuf, [jnp.minimum(dest, 127)],
                                       vals, mask=keep & (dest < cap))
                    carry_cs = carry_cs + _splat_i(jnp.max(kc))
                m_new = jnp.minimum(carry_cs, cap)
                new_ps = [pixbuf[pl.ds(c * _L, _L)] for c in range(nch)]
                return tuple(new_ps) + (m_new,)

            init = tuple(_iota() + (c * _L) for c in range(nch)) + (
                _splat_i(64),)
            fin = lax.fori_loop(0, 6, level, init)
            ps = list(fin[:nch])
            msplat = fin[nch]

            smax = _splat_f(_MASKVAL)
            svals = []
            for c in range(4):
                zv = _iota() + (c * _L)
                idx = jnp.minimum(ps[c], t_src - 1)
                sc = plsc.load_gather(srow, [idx])
                sc = jnp.where(zv < msplat, sc, _splat_f(_MASKVAL))
                svals.append(sc)
                smax = jnp.maximum(smax, sc)
                idxbuf[pl.ds(c * _L, _L)] = idx + (row // t_src) * t_src
            mx = _splat_f(jnp.max(smax))
            denom = _splat_f(0.0)
            evals = []
            for c in range(4):
                zv = _iota() + (c * _L)
                e = jnp.where(zv < msplat, jnp.exp(svals[c] - mx),
                              _splat_f(0.0))
                evals.append(e)
                denom = denom + e
            dsum = _splat_f(jnp.sum(denom))

            pltpu.async_copy(v_hbm.at[idxbuf], vrows, sem).wait()
            accs = [_splat_f(0.0) for _ in range(h // _L)]
            io = _iota()
            for zc in range(4):
                pvec = evals[zc] / dsum
                for l in range(_L):
                    z = zc * _L + l
                    pz = _splat_f(jnp.sum(
                        jnp.where(io == l, pvec, _splat_f(0.0))))
                    for c in range(h // _L):
                        accs[c] = accs[c] + pz * vrows[z, pl.ds(c * _L, _L)]
            for c in range(h // _L):
                ctxbuf[pl.ds(c * _L, _L)] = accs[c]
            pltpu.sync_copy(ctxbuf, out_hbm.at[row])
            return 0

        lax.fori_loop(0, rows_per, row_body, 0)

    return sc_attend


def kernel(q, k, v):
    n, t, h = q.shape
    s = _scores(q, k).reshape(n * t, t)
    vf = v.reshape(n * t, h)
    ctx = _make_sc_attend(n * t, t, h)(s, vf)
    return ctx.reshape(n, t, h)


# speedup vs baseline: 1.0606x; 1.0606x over previous
import functools

import jax
import jax.numpy as jnp
from jax import lax
from jax.experimental import pallas as pl
from jax.experimental.pallas import tpu as pltpu
from jax.experimental.pallas import tpu_sc as plsc

_L = 16
_INT_MIN = -2147483648
_MASKVAL = -32000.0


def _scores_body(q_ref, k_ref, o_ref):
    o_ref[0] = lax.dot_general(
        q_ref[0], k_ref[0], (((1,), (1,)), ((), ())),
        precision=lax.Precision.HIGHEST, preferred_element_type=jnp.float32)


def _scores(q, k):
    n, t, h = q.shape
    bm = 256
    return pl.pallas_call(
        _scores_body,
        grid=(n, t // bm),
        in_specs=[pl.BlockSpec((1, bm, h), lambda i, j: (i, j, 0)),
                  pl.BlockSpec((1, t, h), lambda i, j: (i, 0, 0))],
        out_specs=pl.BlockSpec((1, bm, t), lambda i, j: (i, j, 0)),
        out_shape=jax.ShapeDtypeStruct((n, t, t), jnp.float32),
    )(q, k)


def _iota():
    return lax.iota(jnp.int32, _L)


def _splat_i(x):
    return jnp.full((_L,), x, dtype=jnp.int32)


def _splat_f(x):
    return jnp.full((_L,), x, dtype=jnp.float32)


def _rhe(x):
    t = x.astype(jnp.int32)
    r = x - t.astype(jnp.float32)
    odd = (t & 1) == 1
    up = (r > 0.5) | ((r == 0.5) & odd)
    return t + up.astype(jnp.int32)


def _sortable(s):
    b = lax.bitcast_convert_type(s, jnp.int32)
    return jnp.where(b < 0, b ^ jnp.int32(0x7FFFFFFF), b)


def _sort16(v):
    b = jnp.int32(_INT_MIN)
    return lax.sort(v ^ b, dimension=0) ^ b


def _bmerge(vs):
    if len(vs) == 1:
        return [_sort16(vs[0])]
    half = len(vs) // 2
    lo = [jnp.minimum(vs[i], vs[i + half]) for i in range(half)]
    hi = [jnp.maximum(vs[i], vs[i + half]) for i in range(half)]
    return _bmerge(lo) + _bmerge(hi)


def _merge(a, b):
    rb = [lax.rev(x, (0,)) for x in reversed(b)]
    return _bmerge(a + rb)


def _sort128(vs):
    runs = [[_sort16(v)] for v in vs]
    while len(runs) > 1:
        runs = [_merge(runs[i], runs[i + 1]) for i in range(0, len(runs), 2)]
    return runs[0]


def _make_sc_attend(total_rows, t_src, h):
    info = plsc.get_sparse_core_info()
    nw = info.num_cores * info.num_subcores
    rows_per = total_rows // nw
    nch = 8
    ech = 17
    mesh = plsc.VectorSubcoreMesh(core_axis_name="c", subcore_axis_name="s")

    @functools.partial(
        pl.kernel, mesh=mesh,
        compiler_params=pltpu.CompilerParams(
            needs_layout_passes=False, use_tc_tiling_on_sc=False),
        out_type=jax.ShapeDtypeStruct((total_rows, h), jnp.float32),
        scratch_types=[
            pltpu.VMEM((t_src,), jnp.float32),
            pltpu.VMEM((t_src,), jnp.float32),
            pltpu.VMEM((128,), jnp.int32),
            pltpu.VMEM((144,), jnp.int32),
            pltpu.VMEM((ech * _L,), jnp.int32),
            pltpu.VMEM((ech * _L,), jnp.int32),
            pltpu.VMEM((16 + ech * _L,), jnp.int32),
            pltpu.VMEM((64,), jnp.int32),
            pltpu.VMEM((64, h), jnp.float32),
            pltpu.VMEM((h,), jnp.float32),
            pltpu.SemaphoreType.DMA,
            pltpu.SemaphoreType.DMA,
            pltpu.SemaphoreType.DMA,
        ],
    )
    def sc_attend(s_hbm, v_hbm, out_hbm, srow_a, srow_b, pixbuf, key0buf,
                  embuf, incbuf, rmbuf, idxbuf, vrows, ctxbuf, sem,
                  sem_a, sem_b):
        wid = lax.axis_index("c") * info.num_subcores + lax.axis_index("s")

        def process_row(row, srow):
            a = lax.rem(row, t_src)
            tsrc = a + 1
            tsrcf = _splat_f(tsrc.astype(jnp.float32))

            def level(it, carry):
                ps = list(carry[:nch])
                msplat = carry[nch]
                itv = _splat_i(it)
                is0 = itv == 0
                is5 = itv == 5
                wsbase = _splat_i(64 << it).astype(jnp.float32)
                ws = jnp.where(is0, _splat_f(64.0),
                               jnp.minimum(tsrcf, wsbase))
                ratio = tsrcf / ws
                factor = jnp.where(is5, _splat_f(1.0), _splat_f(1.5))
                zm1 = jnp.where(is0, _splat_i(95), _splat_i(127))
                cap = jnp.where(is5, _splat_i(64), _splat_i(128))
                th1_idx = jnp.where(is0, _splat_i(65),
                                    jnp.where(is5, _splat_i(64),
                                              _splat_i(32)))
                ws_new = jnp.minimum(tsrcf, ws * 2.0)
                scale = ws_new / ws

                xt = ws / tsrcf * 64.0 * factor
                bound = jnp.minimum((ws - 1.0).astype(jnp.int32), zm1)
                tks = jnp.minimum(jnp.maximum(_rhe(xt), 1), bound)

                io = _iota()
                key0s = []
                for c in range(nch):
                    zv = io + (c * _L)
                    pf = ps[c].astype(jnp.float32)
                    txs = jnp.minimum(_rhe(pf * ratio), t_src - 1)
                    sc = plsc.load_gather(srow, [txs])
                    sc = jnp.where(zv < msplat, sc, _splat_f(_MASKVAL))
                    k0 = _sortable(sc)
                    key0buf[pl.ds(_L + c * _L, _L)] = k0
                    key0s.append(k0)
                keys = []
                carry_rf = _splat_i(0)
                for c in range(nch):
                    zv = io + (c * _L)
                    prev = key0buf[pl.ds(_L - 1 + c * _L, _L)]
                    chg = (zv == 0) | (key0s[c] != prev)
                    tt = jnp.where(chg, zv, _splat_i(0))
                    rf = jnp.maximum(plsc.cummax(tt), carry_rf)
                    carry_rf = _splat_i(jnp.max(rf))
                    kk = key0s[c] - (zv - rf)
                    ph = is0 & (zv >= 96)
                    keys.append(jnp.where(ph, _splat_i(_INT_MIN), kk))

                skeys = _sort128(keys)
                th2_idx = jnp.minimum(_splat_i(128) - tks, 127)
                th1 = _splat_i(0)
                th2 = _splat_i(0)
                for c in range(nch):
                    zv = io + (c * _L)
                    th1 = th1 + jnp.where(zv == th1_idx, skeys[c],
                                          _splat_i(0))
                    th2 = th2 + jnp.where(zv == th2_idx, skeys[c],
                                          _splat_i(0))
                th1 = _splat_i(jnp.sum(th1))
                th2 = _splat_i(jnp.sum(th2))
                tks_pos = tks > 0

                cnt0 = _rhe(scale)
                inc_head = ((tks > msplat) & (cnt0 > 1)).astype(jnp.int32)
                io = _iota()
                embuf[pl.ds(0, _L)] = jnp.where(io == 1, 1, 0)
                head_inc = jnp.where(
                    io == 0, _splat_i(1),
                    jnp.where(io == 1, inc_head, _splat_i(0)))
                incbuf[pl.ds(0, _L)] = head_inc
                incbuf[pl.ds(256, _L)] = _splat_i(0)
                for c in range(nch):
                    zv = _iota() + (c * _L)
                    validc = zv < msplat
                    sel1 = (keys[c] >= th1) & validc
                    sel2 = (keys[c] >= th2) & tks_pos & validc
                    pf = ps[c].astype(jnp.float32)
                    c0 = _rhe(pf * scale)
                    pe = _rhe((pf + 1.0) * scale)
                    c1 = c0 + ((pe - c0 > 1) & sel2).astype(jnp.int32)
                    slot0 = 2 + 2 * zv
                    plsc.store_scatter(embuf, [slot0], c0)
                    plsc.store_scatter(embuf, [slot0 + 1], c1)
                    plsc.store_scatter(incbuf, [slot0],
                                       sel1.astype(jnp.int32))
                    plsc.store_scatter(incbuf, [slot0 + 1],
                                       sel2.astype(jnp.int32))

                for c in range(nch):
                    pixbuf[pl.ds(c * _L, _L)] = _splat_i(0)
                carry_rm = _splat_i(-1)
                carry_cs = _splat_i(0)
                for j in range(ech):
                    vals = embuf[pl.ds(j * _L, _L)]
                    incs = incbuf[pl.ds(j * _L, _L)]
                    mv = jnp.where(incs > 0, vals, _splat_i(-1))
                    rm = jnp.maximum(plsc.cummax(mv), carry_rm)
                    rmbuf[pl.ds(_L + j * _L, _L)] = rm
                    excl = rmbuf[pl.ds(_L - 1 + j * _L, _L)]
                    if j == 0:
                        excl = jnp.where(io == 0, _splat_i(-1), excl)
                    carry_rm = _splat_i(jnp.max(rm))
                    keep = (incs > 0) & (vals > excl)
                    ki = keep.astype(jnp.int32)
                    kc = plsc.cumsum(ki)
                    dest = carry_cs + kc - ki
                    plsc.store_scatter(pixbuf, [jnp.minimum(dest, 127)],
                                       vals, mask=keep & (dest < cap))
                    carry_cs = carry_cs + _splat_i(jnp.max(kc))
                m_new = jnp.minimum(carry_cs, cap)
                new_ps = [pixbuf[pl.ds(c * _L, _L)] for c in range(nch)]
                return tuple(new_ps) + (m_new,)

            init = tuple(_iota() + (c * _L) for c in range(nch)) + (
                _splat_i(64),)
            fin = lax.fori_loop(0, 6, level, init)
            ps = list(fin[:nch])
            msplat = fin[nch]

            smax = _splat_f(_MASKVAL)
            svals = []
            for c in range(4):
                zv = _iota() + (c * _L)
                idx = jnp.minimum(ps[c], t_src - 1)
                sc = plsc.load_gather(srow, [idx])
                sc = jnp.where(zv < msplat, sc, _splat_f(_MASKVAL))
                svals.append(sc)
                smax = jnp.maximum(smax, sc)
                idxbuf[pl.ds(c * _L, _L)] = idx + (row // t_src) * t_src
            mx = _splat_f(jnp.max(smax))
            denom = _splat_f(0.0)
            evals = []
            for c in range(4):
                zv = _iota() + (c * _L)
                e = jnp.where(zv < msplat, jnp.exp(svals[c] - mx),
                              _splat_f(0.0))
                evals.append(e)
                denom = denom + e
            dsum = _splat_f(jnp.sum(denom))

            pltpu.async_copy(v_hbm.at[idxbuf], vrows, sem).wait()
            accs = [_splat_f(0.0) for _ in range(h // _L)]
            io = _iota()
            for zc in range(4):
                pvec = evals[zc] / dsum
                for l in range(_L):
                    z = zc * _L + l
                    pz = _splat_f(jnp.sum(
                        jnp.where(io == l, pvec, _splat_f(0.0))))
                    for c in range(h // _L):
                        accs[c] = accs[c] + pz * vrows[z, pl.ds(c * _L, _L)]
            for c in range(h // _L):
                ctxbuf[pl.ds(c * _L, _L)] = accs[c]
            pltpu.sync_copy(ctxbuf, out_hbm.at[row])

        def rowid(r):
            return r * nw + wid

        pltpu.async_copy(s_hbm.at[rowid(0)], srow_a, sem_a)

        def pair_body(j, _):
            r0 = 2 * j
            pltpu.async_copy(s_hbm.at[rowid(r0 + 1)], srow_b, sem_b)
            pltpu.make_async_copy(s_hbm.at[rowid(r0)], srow_a, sem_a).wait()
            process_row(rowid(r0), srow_a)
            rnext = jnp.minimum(r0 + 2, rows_per - 2)
            pltpu.async_copy(s_hbm.at[rowid(rnext)], srow_a, sem_a)
            pltpu.make_async_copy(s_hbm.at[rowid(r0 + 1)], srow_b,
                                  sem_b).wait()
            process_row(rowid(r0 + 1), srow_b)
            return 0

        lax.fori_loop(0, rows_per // 2, pair_body, 0)
        pltpu.make_async_copy(s_hbm.at[rowid(0)], srow_a, sem_a).wait()

    return sc_attend


def kernel(q, k, v):
    n, t, h = q.shape
    s = _scores(q, k).reshape(n * t, t)
    vf = v.reshape(n * t, h)
    ctx = _make_sc_attend(n * t, t, h)(s, vf)
    return ctx.reshape(n, t, h)
